# baseline (device time: 56172 ns/iter reference)
import jax
import jax.numpy as jnp
from jax import lax
from jax.experimental import pallas as pl
from jax.experimental.pallas import tpu as pltpu

N_DEV = 8
GENS = (1, 3, 4)


def kernel(A, B):
    m, _ = A.shape
    _, n = B.shape
    f32 = jnp.float32
    bf16 = jnp.bfloat16
    third = m // 3

    def body(a_ref, b_ref, out_ref, sb0, sb1, sb2, rb0, rb1, rb2, ag_buf,
             rs_send, rs_recv, ag_send, ag_recv):
        me = lax.axis_index("i")
        bit0 = me & 1
        bit1 = (me >> 1) & 1
        bit2 = (me >> 2) & 1
        c = (bit0 ^ bit1, bit1, bit2)

        barrier = pltpu.get_barrier_semaphore()
        for g in GENS:
            pl.semaphore_signal(
                barrier, inc=1,
                device_id=(me ^ g,), device_id_type=pl.DeviceIdType.MESH,
            )
        pl.semaphore_wait(barrier, 3)

        sbufs = [sb0, sb1, sb2]
        rbufs = [rb0, rb1, rb2]
        sizes = [third // 2, third // 4, third // 8]

        def mm(r0, rows):
            return jnp.dot(a_ref[pl.ds(r0, rows), :].astype(bf16),
                           b_ref[:, :].astype(bf16),
                           preferred_element_type=f32)

        def make_rs(t, s):
            return pltpu.make_async_remote_copy(
                src_ref=sbufs[s].at[t],
                dst_ref=rbufs[s].at[t],
                send_sem=rs_send.at[t, s],
                recv_sem=rs_recv.at[t, s],
                device_id=(me ^ GENS[(t + s) % 3],),
                device_id_type=pl.DeviceIdType.MESH,
            )

        starts = []
        rdmas = [None, None, None]
        for t in range(3):
            ck = c[t]
            send = t * third + (1 - ck) * sizes[0]
            sbufs[0][t, :, :] = mm(send, sizes[0]).astype(bf16)
            rdmas[t] = make_rs(t, 0)
            rdmas[t].start()
            starts.append(t * third + ck * sizes[0])
        for t in range(3):
            out_ref[pl.ds(starts[t], sizes[0]), :] = mm(starts[t], sizes[0])

        for s in (1, 2):
            new_rdmas = [None, None, None]
            for t in range(3):
                rdmas[t].wait()
                acc = (out_ref[pl.ds(starts[t], sizes[s - 1]), :]
                       + rbufs[s - 1][t].astype(f32))
                ck = c[(t + s) % 3]
                keep = starts[t] + ck * sizes[s]
                send = starts[t] + (1 - ck) * sizes[s]
                out_ref[pl.ds(starts[t], sizes[s - 1]), :] = acc
                sbufs[s][t, :, :] = out_ref[pl.ds(send, sizes[s]), :].astype(bf16)
                new_rdmas[t] = make_rs(t, s)
                new_rdmas[t].start()
                starts[t] = keep
            rdmas = new_rdmas

        for t in range(3):
            rdmas[t].wait()
            z = (out_ref[pl.ds(starts[t], sizes[2]), :]
                 + rbufs[2][t].astype(f32))
            g = 0.5 * z * (1.0 + jnp.tanh(
                0.7978845608 * (z + 0.044715 * z * z * z)))
            ag_buf[pl.ds(starts[t], sizes[2]), :] = g.astype(bf16)

        size = sizes[2]
        for s in range(3):
            rdmas = [None, None, None]
            for t in range(3):
                rdmas[t] = pltpu.make_async_remote_copy(
                    src_ref=ag_buf.at[pl.ds(starts[t], size)],
                    dst_ref=ag_buf.at[pl.ds(starts[t], size)],
                    send_sem=ag_send.at[t, s],
                    recv_sem=ag_recv.at[t, s],
                    device_id=(me ^ GENS[(t + 2 - s) % 3],),
                    device_id_type=pl.DeviceIdType.MESH,
                )
                rdmas[t].start()
            for t in range(3):
                rdmas[t].wait()
                starts[t] = starts[t] - c[(t + 2 - s) % 3] * size
            size *= 2

        out_ref[:, :] = ag_buf[:, :].astype(f32)

    return pl.pallas_call(
        body,
        out_shape=jax.ShapeDtypeStruct((m, n), f32),
        in_specs=[
            pl.BlockSpec(memory_space=pltpu.VMEM),
            pl.BlockSpec(memory_space=pltpu.VMEM),
        ],
        out_specs=pl.BlockSpec(memory_space=pltpu.VMEM),
        scratch_shapes=[
            pltpu.VMEM((3, third // 2, n), bf16),
            pltpu.VMEM((3, third // 4, n), bf16),
            pltpu.VMEM((3, third // 8, n), bf16),
            pltpu.VMEM((3, third // 2, n), bf16),
            pltpu.VMEM((3, third // 4, n), bf16),
            pltpu.VMEM((3, third // 8, n), bf16),
            pltpu.VMEM((m, n), bf16),
            pltpu.SemaphoreType.DMA((3, 3)),
            pltpu.SemaphoreType.DMA((3, 3)),
            pltpu.SemaphoreType.DMA((3, 3)),
            pltpu.SemaphoreType.DMA((3, 3)),
        ],
        compiler_params=pltpu.CompilerParams(collective_id=0),
    )(A, B)


# device time: 53854 ns/iter; 1.0430x vs baseline; 1.0430x over previous
import jax
import jax.numpy as jnp
from jax import lax
from jax.experimental import pallas as pl
from jax.experimental.pallas import tpu as pltpu

N_DEV = 8
GENS = (1, 3, 4)


def kernel(A, B):
    m, _ = A.shape
    _, n = B.shape
    f32 = jnp.float32
    bf16 = jnp.bfloat16
    third = m // 3
    h0 = third // 2
    h1 = third // 4

    def body(a_ref, b_ref, out_ref, sb0, sb1, sb2, rb0, rb1, rb2, ag_buf,
             rs_send, rs_recv, ag_send, ag_recv):
        me = lax.axis_index("i")
        bit0 = me & 1
        bit1 = (me >> 1) & 1
        bit2 = (me >> 2) & 1
        c = (bit0 ^ bit1, bit1, bit2)

        barrier = pltpu.get_barrier_semaphore()
        for g in GENS:
            pl.semaphore_signal(
                barrier, inc=1,
                device_id=(me ^ g,), device_id_type=pl.DeviceIdType.MESH,
            )
        pl.semaphore_wait(barrier, 3)

        sbufs = [sb0, sb1, sb2]
        rbufs = [rb0, rb1, rb2]

        def mm(r0, rows):
            return jnp.dot(a_ref[pl.ds(r0, rows), :].astype(bf16),
                           b_ref[:, :].astype(bf16),
                           preferred_element_type=f32)

        def make_rs(t, s):
            return pltpu.make_async_remote_copy(
                src_ref=sbufs[s].at[t],
                dst_ref=rbufs[s].at[t],
                send_sem=rs_send.at[t, s],
                recv_sem=rs_recv.at[t, s],
                device_id=(me ^ GENS[(t + s) % 3],),
                device_id_type=pl.DeviceIdType.MESH,
            )

        starts = []
        rdmas = [None, None, None]
        for t in range(3):
            ck = c[t]
            send = t * third + (1 - ck) * h0
            sbufs[0][t, :, :] = mm(send, h0).astype(bf16)
            rdmas[t] = make_rs(t, 0)
            rdmas[t].start()
            starts.append(t * third + ck * h0)
        for t in range(3):
            out_ref[pl.ds(starts[t], h0), :] = mm(starts[t], h0)

        new_rdmas = [None, None, None]
        for t in range(3):
            rdmas[t].wait()
            acc = out_ref[pl.ds(starts[t], h0), :] + rbufs[0][t].astype(f32)
            ck = c[(t + 1) % 3]
            out_ref[pl.ds(starts[t], h0), :] = acc
            send = starts[t] + (1 - ck) * h1
            sbufs[1][t, :, :] = out_ref[pl.ds(send, h1), :].astype(bf16)
            new_rdmas[t] = make_rs(t, 1)
            new_rdmas[t].start()
            starts[t] = starts[t] + ck * h1
        rdmas = new_rdmas

        new_rdmas = [None, None, None]
        for t in range(3):
            rdmas[t].wait()
            acc = out_ref[pl.ds(starts[t], h1), :] + rbufs[1][t].astype(f32)
            out_ref[pl.ds(starts[t], h1), :] = acc
            sbufs[2][t, :, :] = acc.astype(bf16)
            new_rdmas[t] = make_rs(t, 2)
            new_rdmas[t].start()
        rdmas = new_rdmas

        pending = []
        for t in range(3):
            rdmas[t].wait()
            z = out_ref[pl.ds(starts[t], h1), :] + rbufs[2][t].astype(f32)
            g = 0.5 * z * (1.0 + jnp.tanh(
                0.7978845608 * (z + 0.044715 * z * z * z)))
            ag_buf[pl.ds(starts[t], h1), :] = g.astype(bf16)
            pending.append((starts[t], h1))

        size = h1
        for s in range(2):
            rdmas = [None, None, None]
            for t in range(3):
                rdmas[t] = pltpu.make_async_remote_copy(
                    src_ref=ag_buf.at[pl.ds(starts[t], size)],
                    dst_ref=ag_buf.at[pl.ds(starts[t], size)],
                    send_sem=ag_send.at[t, s],
                    recv_sem=ag_recv.at[t, s],
                    device_id=(me ^ GENS[(t + 1 - s) % 3],),
                    device_id_type=pl.DeviceIdType.MESH,
                )
                rdmas[t].start()
            for r0, rows in pending:
                out_ref[pl.ds(r0, rows), :] = ag_buf[pl.ds(r0, rows), :].astype(f32)
            pending = []
            for t in range(3):
                rdmas[t].wait()
                ck = c[(t + 1 - s) % 3]
                new_start = starts[t] - ck * size
                pending.append((new_start + (1 - ck) * size, size))
                starts[t] = new_start
            size *= 2
        for r0, rows in pending:
            out_ref[pl.ds(r0, rows), :] = ag_buf[pl.ds(r0, rows), :].astype(f32)

    return pl.pallas_call(
        body,
        out_shape=jax.ShapeDtypeStruct((m, n), f32),
        in_specs=[
            pl.BlockSpec(memory_space=pltpu.VMEM),
            pl.BlockSpec(memory_space=pltpu.VMEM),
        ],
        out_specs=pl.BlockSpec(memory_space=pltpu.VMEM),
        scratch_shapes=[
            pltpu.VMEM((3, h0, n), bf16),
            pltpu.VMEM((3, h1, n), bf16),
            pltpu.VMEM((3, h1, n), bf16),
            pltpu.VMEM((3, h0, n), bf16),
            pltpu.VMEM((3, h1, n), bf16),
            pltpu.VMEM((3, h1, n), bf16),
            pltpu.VMEM((m, n), bf16),
            pltpu.SemaphoreType.DMA((3, 3)),
            pltpu.SemaphoreType.DMA((3, 3)),
            pltpu.SemaphoreType.DMA((3, 2)),
            pltpu.SemaphoreType.DMA((3, 2)),
        ],
        compiler_params=pltpu.CompilerParams(collective_id=0),
    )(A, B)


# device time: 49648 ns/iter; 1.1314x vs baseline; 1.0847x over previous
import jax
import jax.numpy as jnp
from jax import lax
from jax.experimental import pallas as pl
from jax.experimental.pallas import tpu as pltpu

N_DEV = 8
GENS = (1, 3, 4)


def kernel(A, B):
    m, _ = A.shape
    _, n = B.shape
    f32 = jnp.float32
    bf16 = jnp.bfloat16
    third = m // 3
    R = third // 2
    h0 = R // 2
    h1 = R // 4
    insts = [(t, u) for u in range(2) for t in range(3)]
    jidx = {(t, u): t * 2 + u for t, u in insts}

    def body(a_ref, b_ref, out_ref, sb0, sb1, sb2, rb0, rb1, rb2, ag_buf,
             rs_send, rs_recv, ag_send, ag_recv):
        me = lax.axis_index("i")
        bit0 = me & 1
        bit1 = (me >> 1) & 1
        bit2 = (me >> 2) & 1
        c = (bit0 ^ bit1, bit1, bit2)

        barrier = pltpu.get_barrier_semaphore()
        for g in GENS:
            pl.semaphore_signal(
                barrier, inc=1,
                device_id=(me ^ g,), device_id_type=pl.DeviceIdType.MESH,
            )
        pl.semaphore_wait(barrier, 3)

        sbufs = [sb0, sb1, sb2]
        rbufs = [rb0, rb1, rb2]

        def mm(r0, rows):
            return jnp.dot(a_ref[pl.ds(r0, rows), :].astype(bf16),
                           b_ref[:, :].astype(bf16),
                           preferred_element_type=f32)

        def make_rs(t, u, s):
            j = jidx[(t, u)]
            return pltpu.make_async_remote_copy(
                src_ref=sbufs[s].at[j],
                dst_ref=rbufs[s].at[j],
                send_sem=rs_send.at[j, s],
                recv_sem=rs_recv.at[j, s],
                device_id=(me ^ GENS[(t + s) % 3],),
                device_id_type=pl.DeviceIdType.MESH,
            )

        starts = {}
        rdmas = {}
        for t, u in insts:
            j = jidx[(t, u)]
            base = t * third + u * R
            ck = c[t]
            send = base + (1 - ck) * h0
            sbufs[0][j, :, :] = mm(send, h0).astype(bf16)
            rdmas[(t, u)] = make_rs(t, u, 0)
            rdmas[(t, u)].start()
            starts[(t, u)] = base + ck * h0
        for t, u in insts:
            out_ref[pl.ds(starts[(t, u)], h0), :] = mm(starts[(t, u)], h0)

        new_rdmas = {}
        for t, u in insts:
            j = jidx[(t, u)]
            rdmas[(t, u)].wait()
            acc = (out_ref[pl.ds(starts[(t, u)], h0), :]
                   + rbufs[0][j].astype(f32))
            ck = c[(t + 1) % 3]
            out_ref[pl.ds(starts[(t, u)], h0), :] = acc
            send = starts[(t, u)] + (1 - ck) * h1
            sbufs[1][j, :, :] = out_ref[pl.ds(send, h1), :].astype(bf16)
            new_rdmas[(t, u)] = make_rs(t, u, 1)
            new_rdmas[(t, u)].start()
            starts[(t, u)] = starts[(t, u)] + ck * h1
        rdmas = new_rdmas

        new_rdmas = {}
        for t, u in insts:
            j = jidx[(t, u)]
            rdmas[(t, u)].wait()
            acc = (out_ref[pl.ds(starts[(t, u)], h1), :]
                   + rbufs[1][j].astype(f32))
            out_ref[pl.ds(starts[(t, u)], h1), :] = acc
            sbufs[2][j, :, :] = acc.astype(bf16)
            new_rdmas[(t, u)] = make_rs(t, u, 2)
            new_rdmas[(t, u)].start()
        rdmas = new_rdmas

        pending = []
        for t, u in insts:
            j = jidx[(t, u)]
            rdmas[(t, u)].wait()
            z = (out_ref[pl.ds(starts[(t, u)], h1), :]
                 + rbufs[2][j].astype(f32))
            g = 0.5 * z * (1.0 + jnp.tanh(
                0.7978845608 * (z + 0.044715 * z * z * z)))
            ag_buf[pl.ds(starts[(t, u)], h1), :] = g.astype(bf16)
            pending.append((starts[(t, u)], h1))

        size = h1
        for s in range(2):
            rdmas = {}
            for t, u in insts:
                j = jidx[(t, u)]
                rdmas[(t, u)] = pltpu.make_async_remote_copy(
                    src_ref=ag_buf.at[pl.ds(starts[(t, u)], size)],
                    dst_ref=ag_buf.at[pl.ds(starts[(t, u)], size)],
                    send_sem=ag_send.at[j, s],
                    recv_sem=ag_recv.at[j, s],
                    device_id=(me ^ GENS[(t + 1 - s) % 3],),
                    device_id_type=pl.DeviceIdType.MESH,
                )
                rdmas[(t, u)].start()
            for r0, rows in pending:
                out_ref[pl.ds(r0, rows), :] = ag_buf[pl.ds(r0, rows), :].astype(f32)
            pending = []
            for t, u in insts:
                rdmas[(t, u)].wait()
                ck = c[(t + 1 - s) % 3]
                new_start = starts[(t, u)] - ck * size
                pending.append((new_start + (1 - ck) * size, size))
                starts[(t, u)] = new_start
            size *= 2
        for r0, rows in pending:
            out_ref[pl.ds(r0, rows), :] = ag_buf[pl.ds(r0, rows), :].astype(f32)

    return pl.pallas_call(
        body,
        out_shape=jax.ShapeDtypeStruct((m, n), f32),
        in_specs=[
            pl.BlockSpec(memory_space=pltpu.VMEM),
            pl.BlockSpec(memory_space=pltpu.VMEM),
        ],
        out_specs=pl.BlockSpec(memory_space=pltpu.VMEM),
        scratch_shapes=[
            pltpu.VMEM((6, h0, n), bf16),
            pltpu.VMEM((6, h1, n), bf16),
            pltpu.VMEM((6, h1, n), bf16),
            pltpu.VMEM((6, h0, n), bf16),
            pltpu.VMEM((6, h1, n), bf16),
            pltpu.VMEM((6, h1, n), bf16),
            pltpu.VMEM((m, n), bf16),
            pltpu.SemaphoreType.DMA((6, 3)),
            pltpu.SemaphoreType.DMA((6, 3)),
            pltpu.SemaphoreType.DMA((6, 2)),
            pltpu.SemaphoreType.DMA((6, 2)),
        ],
        compiler_params=pltpu.CompilerParams(collective_id=0),
    )(A, B)


# device time: 46426 ns/iter; 1.2099x vs baseline; 1.0694x over previous
import jax
import jax.numpy as jnp
from jax import lax
from jax.experimental import pallas as pl
from jax.experimental.pallas import tpu as pltpu

N_DEV = 8
GENS = (1, 3, 4)


def kernel(A, B):
    m, k = A.shape
    _, n = B.shape
    f32 = jnp.float32
    bf16 = jnp.bfloat16
    third = m // 3
    R = third // 2
    h0 = R // 2
    h1 = R // 4
    insts = [(t, u) for u in range(2) for t in range(3)]
    jidx = {(t, u): t * 2 + u for t, u in insts}

    def body(a_ref, b_ref, out_ref, a16, b16, sb0, sb1, sb2,
             rb0, rb1, rb2, ag_buf, rs_send, rs_recv, ag_send, ag_recv):
        me = lax.axis_index("i")
        bit0 = me & 1
        bit1 = (me >> 1) & 1
        bit2 = (me >> 2) & 1
        c = (bit0 ^ bit1, bit1, bit2)

        barrier = pltpu.get_barrier_semaphore()
        for g in GENS:
            pl.semaphore_signal(
                barrier, inc=1,
                device_id=(me ^ g,), device_id_type=pl.DeviceIdType.MESH,
            )
        pl.semaphore_wait(barrier, 3)

        a16[:, :] = a_ref[:, :].astype(bf16)
        b16[:, :] = b_ref[:, :].astype(bf16)

        sbufs = [sb0, sb1, sb2]
        rbufs = [rb0, rb1, rb2]

        def mm(r0, rows):
            return jnp.dot(a16[pl.ds(r0, rows), :], b16[:, :],
                           preferred_element_type=f32)

        def make_rs(t, u, s):
            j = jidx[(t, u)]
            return pltpu.make_async_remote_copy(
                src_ref=sbufs[s].at[j],
                dst_ref=rbufs[s].at[j],
                send_sem=rs_send.at[j, s],
                recv_sem=rs_recv.at[j, s],
                device_id=(me ^ GENS[(t + s) % 3],),
                device_id_type=pl.DeviceIdType.MESH,
            )

        starts = {}
        rdmas = {}
        for t, u in insts:
            j = jidx[(t, u)]
            base = t * third + u * R
            ck = c[t]
            send = base + (1 - ck) * h0
            sbufs[0][j, :, :] = mm(send, h0).astype(bf16)
            rdmas[(t, u)] = make_rs(t, u, 0)
            rdmas[(t, u)].start()
            starts[(t, u)] = base + ck * h0
        for t, u in insts:
            out_ref[pl.ds(starts[(t, u)], h0), :] = mm(starts[(t, u)], h0)

        new_rdmas = {}
        for t, u in insts:
            j = jidx[(t, u)]
            rdmas[(t, u)].wait()
            ck = c[(t + 1) % 3]
            keep = starts[(t, u)] + ck * h1
            send = starts[(t, u)] + (1 - ck) * h1
            acc_send = (out_ref[pl.ds(send, h1), :]
                        + rbufs[0][j, pl.ds((1 - ck) * h1, h1), :].astype(f32))
            sbufs[1][j, :, :] = acc_send.astype(bf16)
            new_rdmas[(t, u)] = make_rs(t, u, 1)
            new_rdmas[(t, u)].start()
            out_ref[pl.ds(keep, h1), :] = (
                out_ref[pl.ds(keep, h1), :]
                + rbufs[0][j, pl.ds(ck * h1, h1), :].astype(f32)
            )
            starts[(t, u)] = keep
        rdmas = new_rdmas

        new_rdmas = {}
        for t, u in insts:
            j = jidx[(t, u)]
            rdmas[(t, u)].wait()
            acc = (out_ref[pl.ds(starts[(t, u)], h1), :]
                   + rbufs[1][j].astype(f32))
            sbufs[2][j, :, :] = acc.astype(bf16)
            new_rdmas[(t, u)] = make_rs(t, u, 2)
            new_rdmas[(t, u)].start()
            out_ref[pl.ds(starts[(t, u)], h1), :] = acc
        rdmas = new_rdmas

        ag0 = {}
        for t, u in insts:
            j = jidx[(t, u)]
            rdmas[(t, u)].wait()
            z = (out_ref[pl.ds(starts[(t, u)], h1), :]
                 + rbufs[2][j].astype(f32))
            g = 0.5 * z * (1.0 + jnp.tanh(
                0.7978845608 * (z + 0.044715 * z * z * z)))
            ag_buf[pl.ds(starts[(t, u)], h1), :] = g.astype(bf16)
            ag0[(t, u)] = pltpu.make_async_remote_copy(
                src_ref=ag_buf.at[pl.ds(starts[(t, u)], h1)],
                dst_ref=ag_buf.at[pl.ds(starts[(t, u)], h1)],
                send_sem=ag_send.at[j, 0],
                recv_sem=ag_recv.at[j, 0],
                device_id=(me ^ GENS[(t + 1) % 3],),
                device_id_type=pl.DeviceIdType.MESH,
            )
            ag0[(t, u)].start()
            out_ref[pl.ds(starts[(t, u)], h1), :] = g

        ag1 = {}
        pending = []
        for t, u in insts:
            j = jidx[(t, u)]
            ag0[(t, u)].wait()
            ck = c[(t + 1) % 3]
            new_start = starts[(t, u)] - ck * h1
            pending.append((new_start + (1 - ck) * h1, h1))
            starts[(t, u)] = new_start
            ag1[(t, u)] = pltpu.make_async_remote_copy(
                src_ref=ag_buf.at[pl.ds(new_start, h0)],
                dst_ref=ag_buf.at[pl.ds(new_start, h0)],
                send_sem=ag_send.at[j, 1],
                recv_sem=ag_recv.at[j, 1],
                device_id=(me ^ GENS[t],),
                device_id_type=pl.DeviceIdType.MESH,
            )
            ag1[(t, u)].start()
        for r0, rows in pending:
            out_ref[pl.ds(r0, rows), :] = ag_buf[pl.ds(r0, rows), :].astype(f32)
        pending = []
        for t, u in insts:
            ag1[(t, u)].wait()
            ck = c[t]
            new_start = starts[(t, u)] - ck * h0
            pending.append((new_start + (1 - ck) * h0, h0))
            starts[(t, u)] = new_start
        for r0, rows in pending:
            out_ref[pl.ds(r0, rows), :] = ag_buf[pl.ds(r0, rows), :].astype(f32)

    return pl.pallas_call(
        body,
        out_shape=jax.ShapeDtypeStruct((m, n), f32),
        in_specs=[
            pl.BlockSpec(memory_space=pltpu.VMEM),
            pl.BlockSpec(memory_space=pltpu.VMEM),
        ],
        out_specs=pl.BlockSpec(memory_space=pltpu.VMEM),
        scratch_shapes=[
            pltpu.VMEM((m, k), bf16),
            pltpu.VMEM((k, n), bf16),
            pltpu.VMEM((6, h0, n), bf16),
            pltpu.VMEM((6, h1, n), bf16),
            pltpu.VMEM((6, h1, n), bf16),
            pltpu.VMEM((6, h0, n), bf16),
            pltpu.VMEM((6, h1, n), bf16),
            pltpu.VMEM((6, h1, n), bf16),
            pltpu.VMEM((m, n), bf16),
            pltpu.SemaphoreType.DMA((6, 3)),
            pltpu.SemaphoreType.DMA((6, 3)),
            pltpu.SemaphoreType.DMA((6, 2)),
            pltpu.SemaphoreType.DMA((6, 2)),
        ],
        compiler_params=pltpu.CompilerParams(collective_id=0),
    )(A, B)
